# baseline (speedup 1.0000x reference)
import functools

import jax
import jax.numpy as jnp
from jax import lax
from jax.experimental import pallas as pl
from jax.experimental.pallas import tpu as pltpu
from jax.experimental.pallas import tpu_sc as plsc

B, N, C, E = 2, 10000, 128, 320000
T = 7
NC, NS = 2, 16
EPT = E // NS
K = 64
GC = 40
GCK = GC * K
NGROUP = 8
EPT_PAD = NGROUP * GCK
NDUMP = N
RPT_LO = 624
RPT_HI = N - (NS - 1) * RPT_LO


def _expand_body(x_ref, w_ref, y_ref):
    xb = x_ref[0]
    for t in range(T):
        y_ref[0, t] = xb * w_ref[0, t]


def _expand(x, weight):
    nb = 5
    blk = N // nb
    return pl.pallas_call(
        _expand_body,
        grid=(B, nb),
        in_specs=[
            pl.BlockSpec((1, blk, C), lambda b, n: (b, n, 0)),
            pl.BlockSpec((1, T, C), lambda b, n: (0, 0, 0)),
        ],
        out_specs=pl.BlockSpec((1, T, blk, C), lambda b, n: (b, 0, n, 0)),
        out_shape=jax.ShapeDtypeStruct((B, T, N, C), jnp.float32),
    )(x, weight)


def _sc_scatter_fn(y_hbm, gidx_hbm, dst_hbm, bias_hbm, out_hbm,
                   gidx0, gidx1, dst0, dst1, rows, accum,
                   semG0, semG1, semG2, semG3,
                   semS0, semS1, semS2, semS3, semI0, semI1):
    c = lax.axis_index("c")
    s = lax.axis_index("s")
    row0 = s * RPT_LO
    semG = (semG0, semG1, semG2, semG3)
    semS = (semS0, semS1, semS2, semS3)

    @pl.when(s < NS - 1)
    def _init_lo():
        pltpu.sync_copy(bias_hbm.at[pl.ds(0, RPT_LO)],
                        accum.at[pl.ds(row0, RPT_LO), :])

    @pl.when(s == NS - 1)
    def _init_hi():
        pltpu.sync_copy(bias_hbm, accum.at[pl.ds(row0, RPT_HI), :])

    pltpu.sync_copy(gidx_hbm.at[c, s, 0], gidx0)
    pltpu.sync_copy(dst_hbm.at[s, 0], dst0)
    pltpu.async_copy(gidx_hbm.at[c, s, 1], gidx1, semI1)
    pltpu.async_copy(dst_hbm.at[s, 1], dst1, semI1)

    plsc.subcore_barrier()

    pltpu.async_copy(y_hbm.at[gidx0.at[pl.ds(0, K)]], rows.at[0], semG[0])
    pltpu.async_copy(y_hbm.at[gidx0.at[pl.ds(K, K)]], rows.at[1], semG[1])

    def process_group(g, gs):
        gidx_s = gidx0 if gs == 0 else gidx1
        gidx_o = gidx1 if gs == 0 else gidx0
        dst_s = dst0 if gs == 0 else dst1
        dst_o = dst1 if gs == 0 else dst0
        semI_next = semI1 if gs == 0 else semI0
        semI_self = semI0 if gs == 0 else semI1

        @pl.when(g + 1 < NGROUP)
        def _prep_next():
            g1 = g + 1
            pltpu.make_async_copy(gidx_hbm.at[c, s, g1], gidx_o,
                                  semI_next).wait()
            pltpu.make_async_copy(dst_hbm.at[s, g1], dst_o,
                                  semI_next).wait()

        def chunk_quad(j, carry):
            for q in range(4):
                jc = j * 4 + q
                q2 = (q + 2) % 4
                pltpu.make_async_copy(
                    y_hbm.at[gidx_s.at[pl.ds(jc * K, K)]], rows.at[q],
                    semG[q]).wait()
                @pl.when(jc < GC - 2)
                def _scat_async():
                    pltpu.async_copy(
                        rows.at[q], accum.at[dst_s.at[pl.ds(jc * K, K)]],
                        semS[q], add=True)

                @pl.when(jc >= GC - 2)
                def _scat_sync():
                    pltpu.sync_copy(
                        rows.at[q], accum.at[dst_s.at[pl.ds(jc * K, K)]],
                        add=True)
                @pl.when(jc >= 2)
                def _drain():
                    pltpu.make_async_copy(
                        rows.at[q2], accum.at[dst_s.at[pl.ds(jc * K, K)]],
                        semS[q2]).wait()
                nj = jc + 2

                @pl.when(nj < GC)
                def _fire_in():
                    pltpu.async_copy(
                        y_hbm.at[gidx_s.at[pl.ds(nj * K, K)]],
                        rows.at[q2], semG[q2])

                @pl.when(jnp.logical_and(nj >= GC, g + 1 < NGROUP))
                def _fire_cross():
                    pltpu.async_copy(
                        y_hbm.at[gidx_o.at[pl.ds((nj - GC) * K, K)]],
                        rows.at[q2], semG[q2])
            return carry

        lax.fori_loop(0, GC // 4, chunk_quad, 0)

        @pl.when(g + 2 < NGROUP)
        def _fire_idx():
            g2 = g + 2
            pltpu.async_copy(gidx_hbm.at[c, s, g2], gidx_s, semI_self)
            pltpu.async_copy(dst_hbm.at[s, g2], dst_s, semI_self)

    def group_pair(i, carry):
        process_group(i * 2, 0)
        process_group(i * 2 + 1, 1)
        return carry

    lax.fori_loop(0, NGROUP // 2, group_pair, 0)

    plsc.subcore_barrier()

    @pl.when(s < NS - 1)
    def _wb_lo():
        pltpu.sync_copy(accum.at[pl.ds(row0, RPT_LO), :],
                        out_hbm.at[pl.ds(c * N + row0, RPT_LO), :])

    @pl.when(s == NS - 1)
    def _wb_hi():
        pltpu.sync_copy(accum.at[pl.ds(row0, RPT_HI), :],
                        out_hbm.at[pl.ds(c * N + row0, RPT_HI), :])


@functools.cache
def _sc_scatter():
    return pl.kernel(
        _sc_scatter_fn,
        out_type=jax.ShapeDtypeStruct((B * N, C), jnp.float32),
        mesh=plsc.VectorSubcoreMesh(core_axis_name="c", subcore_axis_name="s",
                                    num_cores=NC, num_subcores=NS),
        scratch_types=[
            pltpu.VMEM((GCK,), jnp.int32),
            pltpu.VMEM((GCK,), jnp.int32),
            pltpu.VMEM((GCK,), jnp.int32),
            pltpu.VMEM((GCK,), jnp.int32),
            pltpu.VMEM((4, K, C), jnp.float32),
            pltpu.VMEM_SHARED((N + 8, C), jnp.float32),
            pltpu.SemaphoreType.DMA,
            pltpu.SemaphoreType.DMA,
            pltpu.SemaphoreType.DMA,
            pltpu.SemaphoreType.DMA,
            pltpu.SemaphoreType.DMA,
            pltpu.SemaphoreType.DMA,
            pltpu.SemaphoreType.DMA,
            pltpu.SemaphoreType.DMA,
            pltpu.SemaphoreType.DMA,
            pltpu.SemaphoreType.DMA,
        ],
    )


def kernel(x, edge_index, weight, bias):
    y = _expand(x, weight).reshape(B * T * N, C)
    pad = ((0, 0), (0, EPT_PAD - EPT))
    tsrc = edge_index[2] * N + edge_index[0]
    tsrc_p = jnp.pad(tsrc.reshape(NS, EPT), pad)
    gidx_all = jnp.stack([tsrc_p, tsrc_p + T * N]).reshape(2, NS, NGROUP, GCK)
    dst_r = jnp.pad(edge_index[1].reshape(NS, EPT), pad,
                    constant_values=NDUMP).reshape(NS, NGROUP, GCK)
    bias_big = jnp.broadcast_to(bias.reshape(1, C), (RPT_HI, C))
    out = _sc_scatter()(y, gidx_all, dst_r, bias_big)
    return out.reshape(B, N, C)


# speedup vs baseline: 1.4739x; 1.4739x over previous
import functools

import jax
import jax.numpy as jnp
from jax import lax
from jax.experimental import pallas as pl
from jax.experimental.pallas import tpu as pltpu
from jax.experimental.pallas import tpu_sc as plsc

B, N, C, E = 2, 10000, 128, 320000
T = 7
NC, NS = 2, 16
EPT = E // NS
K = 56
GC = 36
GCK = GC * K
NGROUP = 10
EPT_PAD = NGROUP * GCK
NDUMP = N
RPT_LO = 624
RPT_HI = N - (NS - 1) * RPT_LO


def _expand_body(x_ref, w_ref, y_ref):
    xb = x_ref[0]
    for t in range(T):
        y_ref[0, t] = xb * w_ref[0, t]


def _expand(x, weight):
    nb = 5
    blk = N // nb
    return pl.pallas_call(
        _expand_body,
        grid=(B, nb),
        in_specs=[
            pl.BlockSpec((1, blk, C), lambda b, n: (b, n, 0)),
            pl.BlockSpec((1, T, C), lambda b, n: (0, 0, 0)),
        ],
        out_specs=pl.BlockSpec((1, T, blk, C), lambda b, n: (b, 0, n, 0)),
        out_shape=jax.ShapeDtypeStruct((B, T, N, C), jnp.float32),
    )(x, weight)


def _sc_scatter_fn(y_hbm, gidx_hbm, dst_hbm, bias_hbm, out_hbm,
                   gidx0, gidx1, dst0, dst1, rows, accum,
                   semG0, semG1, semG2, semG3,
                   semS0, semS1, semS2, semS3, semI0, semI1):
    c = lax.axis_index("c")
    s = lax.axis_index("s")
    row0 = s * RPT_LO
    semG = (semG0, semG1, semG2, semG3)
    semS = (semS0, semS1, semS2, semS3)

    @pl.when(s < NS - 1)
    def _init_lo():
        pltpu.sync_copy(bias_hbm.at[pl.ds(0, RPT_LO)],
                        accum.at[pl.ds(row0, RPT_LO), :])

    @pl.when(s == NS - 1)
    def _init_hi():
        pltpu.sync_copy(bias_hbm, accum.at[pl.ds(row0, RPT_HI), :])

    pltpu.sync_copy(gidx_hbm.at[c, s, 0], gidx0)
    pltpu.sync_copy(dst_hbm.at[s, 0], dst0)
    pltpu.async_copy(gidx_hbm.at[c, s, 1], gidx1, semI1)
    pltpu.async_copy(dst_hbm.at[s, 1], dst1, semI1)

    plsc.subcore_barrier()

    pltpu.async_copy(y_hbm.at[gidx0.at[pl.ds(0, K)]], rows.at[0], semG[0])
    pltpu.async_copy(y_hbm.at[gidx0.at[pl.ds(K, K)]], rows.at[1], semG[1])

    def process_group(g, gs):
        gidx_s = gidx0 if gs == 0 else gidx1
        gidx_o = gidx1 if gs == 0 else gidx0
        dst_s = dst0 if gs == 0 else dst1
        dst_o = dst1 if gs == 0 else dst0
        semI_next = semI1 if gs == 0 else semI0
        semI_self = semI0 if gs == 0 else semI1

        @pl.when(g + 1 < NGROUP)
        def _prep_next():
            g1 = g + 1
            pltpu.make_async_copy(gidx_hbm.at[c, s, g1], gidx_o,
                                  semI_next).wait()
            pltpu.make_async_copy(dst_hbm.at[s, g1], dst_o,
                                  semI_next).wait()

        def chunk_quad(j, carry):
            for q in range(4):
                jc = j * 4 + q
                q2 = (q + 2) % 4
                pltpu.make_async_copy(
                    y_hbm.at[gidx_s.at[pl.ds(jc * K, K)]], rows.at[q],
                    semG[q]).wait()
                @pl.when(jc < GC - 2)
                def _scat_async():
                    pltpu.async_copy(
                        rows.at[q], accum.at[dst_s.at[pl.ds(jc * K, K)]],
                        semS[q], add=True)

                @pl.when(jc >= GC - 2)
                def _scat_sync():
                    pltpu.sync_copy(
                        rows.at[q], accum.at[dst_s.at[pl.ds(jc * K, K)]],
                        add=True)
                @pl.when(jc >= 2)
                def _drain():
                    pltpu.make_async_copy(
                        rows.at[q2], accum.at[dst_s.at[pl.ds(jc * K, K)]],
                        semS[q2]).wait()
                nj = jc + 2

                @pl.when(nj < GC)
                def _fire_in():
                    pltpu.async_copy(
                        y_hbm.at[gidx_s.at[pl.ds(nj * K, K)]],
                        rows.at[q2], semG[q2])

                @pl.when(jnp.logical_and(nj >= GC, g + 1 < NGROUP))
                def _fire_cross():
                    pltpu.async_copy(
                        y_hbm.at[gidx_o.at[pl.ds((nj - GC) * K, K)]],
                        rows.at[q2], semG[q2])
            return carry

        lax.fori_loop(0, GC // 4, chunk_quad, 0)

        @pl.when(g + 2 < NGROUP)
        def _fire_idx():
            g2 = g + 2
            pltpu.async_copy(gidx_hbm.at[c, s, g2], gidx_s, semI_self)
            pltpu.async_copy(dst_hbm.at[s, g2], dst_s, semI_self)

    def group_pair(i, carry):
        process_group(i * 2, 0)
        process_group(i * 2 + 1, 1)
        return carry

    lax.fori_loop(0, NGROUP // 2, group_pair, 0)

    plsc.subcore_barrier()

    @pl.when(s < NS - 1)
    def _wb_lo():
        pltpu.sync_copy(accum.at[pl.ds(row0, RPT_LO), :],
                        out_hbm.at[pl.ds(c * N + row0, RPT_LO), :])

    @pl.when(s == NS - 1)
    def _wb_hi():
        pltpu.sync_copy(accum.at[pl.ds(row0, RPT_HI), :],
                        out_hbm.at[pl.ds(c * N + row0, RPT_HI), :])


@functools.cache
def _sc_scatter():
    return pl.kernel(
        _sc_scatter_fn,
        out_type=jax.ShapeDtypeStruct((B * N, C), jnp.float32),
        mesh=plsc.VectorSubcoreMesh(core_axis_name="c", subcore_axis_name="s",
                                    num_cores=NC, num_subcores=NS),
        scratch_types=[
            pltpu.VMEM((GCK,), jnp.int32),
            pltpu.VMEM((GCK,), jnp.int32),
            pltpu.VMEM((GCK,), jnp.int32),
            pltpu.VMEM((GCK,), jnp.int32),
            pltpu.VMEM((4, K, C), jnp.float32),
            pltpu.VMEM_SHARED((N + 8, C), jnp.float32),
            pltpu.SemaphoreType.DMA,
            pltpu.SemaphoreType.DMA,
            pltpu.SemaphoreType.DMA,
            pltpu.SemaphoreType.DMA,
            pltpu.SemaphoreType.DMA,
            pltpu.SemaphoreType.DMA,
            pltpu.SemaphoreType.DMA,
            pltpu.SemaphoreType.DMA,
            pltpu.SemaphoreType.DMA,
            pltpu.SemaphoreType.DMA,
        ],
    )


def kernel(x, edge_index, weight, bias):
    y = _expand(x, weight).reshape(B * T * N, C)
    pad = ((0, 0), (0, EPT_PAD - EPT))
    tsrc = edge_index[2] * N + edge_index[0]
    tsrc_p = jnp.pad(tsrc.reshape(NS, EPT), pad)
    gidx_all = jnp.stack([tsrc_p, tsrc_p + T * N]).reshape(2, NS, NGROUP, GCK)
    dst_r = jnp.pad(edge_index[1].reshape(NS, EPT), pad,
                    constant_values=NDUMP).reshape(NS, NGROUP, GCK)
    bias_big = jnp.broadcast_to(bias.reshape(1, C), (RPT_HI, C))
    out = _sc_scatter()(y, gidx_all, dst_r, bias_big)
    return out.reshape(B, N, C)


# speedup vs baseline: 1.6933x; 1.1488x over previous
import functools

import jax
import jax.numpy as jnp
from jax import lax
from jax.experimental import pallas as pl
from jax.experimental.pallas import tpu as pltpu
from jax.experimental.pallas import tpu_sc as plsc

B, N, C, E = 2, 10000, 128, 320000
T = 7
NC, NS = 2, 16
EPT = E // NS
K = 40
GC = 100
GCK = GC * K
NGROUP = EPT // GCK
RPT_LO = 624
RPT_HI = N - (NS - 1) * RPT_LO


def _expand_body(x_ref, w_ref, y_ref):
    xb = x_ref[0]
    for t in range(T):
        y_ref[0, t] = xb * w_ref[0, t]


def _expand(x, weight):
    nb = 2
    blk = N // nb
    return pl.pallas_call(
        _expand_body,
        grid=(B, nb),
        in_specs=[
            pl.BlockSpec((1, blk, C), lambda b, n: (b, n, 0)),
            pl.BlockSpec((1, T, C), lambda b, n: (0, 0, 0)),
        ],
        out_specs=pl.BlockSpec((1, T, blk, C), lambda b, n: (b, 0, n, 0)),
        out_shape=jax.ShapeDtypeStruct((B, T, N, C), jnp.float32),
    )(x, weight)


def _sc_scatter_fn(y_hbm, gidx_hbm, dst_hbm, bias_hbm, out_hbm,
                   gidx0, gidx1, dst0, dst1, rows, accum,
                   semG0, semG1, semG2, semG3,
                   semS0, semS1, semS2, semS3, semI0, semI1):
    c = lax.axis_index("c")
    s = lax.axis_index("s")
    row0 = s * RPT_LO
    semG = (semG0, semG1, semG2, semG3)
    semS = (semS0, semS1, semS2, semS3)

    @pl.when(s < NS - 1)
    def _init_lo():
        pltpu.sync_copy(bias_hbm.at[pl.ds(0, RPT_LO)],
                        accum.at[pl.ds(row0, RPT_LO), :])

    @pl.when(s == NS - 1)
    def _init_hi():
        pltpu.sync_copy(bias_hbm, accum.at[pl.ds(row0, RPT_HI), :])

    pltpu.sync_copy(gidx_hbm.at[c, s, 0], gidx0)
    pltpu.sync_copy(dst_hbm.at[s, 0], dst0)
    pltpu.async_copy(gidx_hbm.at[c, s, 1], gidx1, semI1)
    pltpu.async_copy(dst_hbm.at[s, 1], dst1, semI1)

    plsc.subcore_barrier()

    pltpu.async_copy(y_hbm.at[gidx0.at[pl.ds(0, K)]], rows.at[0], semG[0])
    pltpu.async_copy(y_hbm.at[gidx0.at[pl.ds(K, K)]], rows.at[1], semG[1])

    def process_group(g, gs):
        gidx_s = gidx0 if gs == 0 else gidx1
        gidx_o = gidx1 if gs == 0 else gidx0
        dst_s = dst0 if gs == 0 else dst1
        dst_o = dst1 if gs == 0 else dst0
        semI_next = semI1 if gs == 0 else semI0
        semI_self = semI0 if gs == 0 else semI1

        @pl.when(g + 1 < NGROUP)
        def _prep_next():
            g1 = g + 1
            pltpu.make_async_copy(gidx_hbm.at[c, s, g1], gidx_o,
                                  semI_next).wait()
            pltpu.make_async_copy(dst_hbm.at[s, g1], dst_o,
                                  semI_next).wait()

        def chunk_quad(j, carry):
            for q in range(4):
                jc = j * 4 + q
                q2 = (q + 2) % 4
                pltpu.make_async_copy(
                    y_hbm.at[gidx_s.at[pl.ds(jc * K, K)]], rows.at[q],
                    semG[q]).wait()
                @pl.when(jc < GC - 2)
                def _scat_async():
                    pltpu.async_copy(
                        rows.at[q], accum.at[dst_s.at[pl.ds(jc * K, K)]],
                        semS[q], add=True)

                @pl.when(jc >= GC - 2)
                def _scat_sync():
                    pltpu.sync_copy(
                        rows.at[q], accum.at[dst_s.at[pl.ds(jc * K, K)]],
                        add=True)
                @pl.when(jc >= 2)
                def _drain():
                    pltpu.make_async_copy(
                        rows.at[q2], accum.at[dst_s.at[pl.ds(jc * K, K)]],
                        semS[q2]).wait()
                nj = jc + 2

                @pl.when(nj < GC)
                def _fire_in():
                    pltpu.async_copy(
                        y_hbm.at[gidx_s.at[pl.ds(nj * K, K)]],
                        rows.at[q2], semG[q2])

                @pl.when(jnp.logical_and(nj >= GC, g + 1 < NGROUP))
                def _fire_cross():
                    pltpu.async_copy(
                        y_hbm.at[gidx_o.at[pl.ds((nj - GC) * K, K)]],
                        rows.at[q2], semG[q2])
            return carry

        lax.fori_loop(0, GC // 4, chunk_quad, 0)

        @pl.when(g + 2 < NGROUP)
        def _fire_idx():
            g2 = g + 2
            pltpu.async_copy(gidx_hbm.at[c, s, g2], gidx_s, semI_self)
            pltpu.async_copy(dst_hbm.at[s, g2], dst_s, semI_self)

    process_group(0, 0)

    def group_pair(i, carry):
        process_group(i * 2 + 1, 1)
        process_group(i * 2 + 2, 0)
        return carry

    lax.fori_loop(0, (NGROUP - 1) // 2, group_pair, 0)

    plsc.subcore_barrier()

    @pl.when(s < NS - 1)
    def _wb_lo():
        pltpu.sync_copy(accum.at[pl.ds(row0, RPT_LO), :],
                        out_hbm.at[pl.ds(c * N + row0, RPT_LO), :])

    @pl.when(s == NS - 1)
    def _wb_hi():
        pltpu.sync_copy(accum.at[pl.ds(row0, RPT_HI), :],
                        out_hbm.at[pl.ds(c * N + row0, RPT_HI), :])


@functools.cache
def _sc_scatter():
    return pl.kernel(
        _sc_scatter_fn,
        out_type=jax.ShapeDtypeStruct((B * N, C), jnp.float32),
        mesh=plsc.VectorSubcoreMesh(core_axis_name="c", subcore_axis_name="s",
                                    num_cores=NC, num_subcores=NS),
        scratch_types=[
            pltpu.VMEM((GCK,), jnp.int32),
            pltpu.VMEM((GCK,), jnp.int32),
            pltpu.VMEM((GCK,), jnp.int32),
            pltpu.VMEM((GCK,), jnp.int32),
            pltpu.VMEM((4, K, C), jnp.float32),
            pltpu.VMEM_SHARED((N, C), jnp.float32),
            pltpu.SemaphoreType.DMA,
            pltpu.SemaphoreType.DMA,
            pltpu.SemaphoreType.DMA,
            pltpu.SemaphoreType.DMA,
            pltpu.SemaphoreType.DMA,
            pltpu.SemaphoreType.DMA,
            pltpu.SemaphoreType.DMA,
            pltpu.SemaphoreType.DMA,
            pltpu.SemaphoreType.DMA,
            pltpu.SemaphoreType.DMA,
        ],
    )


def kernel(x, edge_index, weight, bias):
    y = _expand(x, weight).reshape(B * T * N, C)
    tsrc = edge_index[2] * N + edge_index[0]
    gidx_all = jnp.stack([tsrc, tsrc + T * N]).reshape(2, NS, NGROUP, GCK)
    dst_r = edge_index[1].reshape(NS, NGROUP, GCK)
    bias_big = jnp.broadcast_to(bias.reshape(1, C), (RPT_HI, C))
    out = _sc_scatter()(y, gidx_all, dst_r, bias_big)
    return out.reshape(B, N, C)
